# baseline (speedup 1.0000x reference)
import jax
import jax.numpy as jnp
import numpy as np
from jax.experimental import pallas as pl
from jax.experimental.pallas import tpu as pltpu

_G = 2.0
_EPS = 1e-7
_NSLAB = 4


def _atan(x):
    s = jnp.sign(x)
    ax = jnp.abs(x)
    big = ax > 2.414213562373095
    mid = ax > 0.4142135623730951
    xr = jnp.where(big, -1.0 / jnp.maximum(ax, 1e-30),
                   jnp.where(mid, (ax - 1.0) / (ax + 1.0), ax))
    z = xr * xr
    p = ((8.05374449538e-2 * z - 1.38776856032e-1) * z
         + 1.99777106478e-1) * z - 3.33329491539e-1
    y = p * z * xr + xr
    y = y + jnp.where(big, np.float32(np.pi / 2),
                      jnp.where(mid, np.float32(np.pi / 4), 0.0))
    return s * y


def _slab_sums(pred, tgt, mask, aw, ah):
    log2e = np.float32(1.4426950408889634)
    ln2 = np.float32(0.6931471805599453)
    sp = jnp.log2(1.0 + jnp.exp2(jnp.abs(pred) * (-log2e))) * ln2
    bce = jnp.maximum(pred, 0.0) - pred * tgt + sp
    colsum = jnp.sum(bce, axis=0, keepdims=True)
    head = bce[0:8, :]
    obj_vec = head[4:5, :]
    h04 = jnp.sum(head[0:5, :], axis=0, keepdims=True)
    cls_vec = (colsum - h04) * mask

    sig = jax.nn.sigmoid(pred[0:4, :])
    px = sig[0:1, :] * _G - (_G - 1.0) / 2.0
    py = sig[1:2, :] * _G - (_G - 1.0) / 2.0
    pw = (sig[2:3, :] * _G) ** 2 * aw
    ph = (sig[3:4, :] * _G) ** 2 * ah
    tx = tgt[0:1, :]
    ty = tgt[1:2, :]
    tw = tgt[2:3, :]
    th = tgt[3:4, :]

    b1x1 = px - pw * 0.5
    b1x2 = px + pw * 0.5
    b1y1 = py - ph * 0.5
    b1y2 = py + ph * 0.5
    b2x1 = tx - tw * 0.5
    b2x2 = tx + tw * 0.5
    b2y1 = ty - th * 0.5
    b2y2 = ty + th * 0.5
    inter = (jnp.maximum(jnp.minimum(b1x2, b2x2) - jnp.maximum(b1x1, b2x1), 0.0)
             * jnp.maximum(jnp.minimum(b1y2, b2y2) - jnp.maximum(b1y1, b2y1), 0.0))
    union = pw * ph + tw * th - inter + _EPS
    iou = inter / union
    cw = jnp.maximum(b1x2, b2x2) - jnp.minimum(b1x1, b2x1)
    ch = jnp.maximum(b1y2, b2y2) - jnp.minimum(b1y1, b2y1)
    c2 = cw * cw + ch * ch + _EPS
    rho2 = (tx - px) ** 2 + (ty - py) ** 2
    ra = tw / (th + _EPS)
    rb = pw / (ph + _EPS)
    v = (4.0 / np.pi ** 2) * _atan((ra - rb) / (1.0 + ra * rb)) ** 2
    alpha = v / (v - iou + 1.0 + _EPS)
    ciou = iou - (rho2 / c2 + v * alpha)
    box_vec = (1.0 - ciou) * mask
    return box_vec, obj_vec, cls_vec


def _loss_kernel(anch_ref, pred_ref, tgt_ref, mask_ref, out_ref, acc_ref):
    i = pl.program_id(0)
    n = pl.num_programs(0)

    @pl.when(i == 0)
    def _():
        acc_ref[...] = jnp.zeros_like(acc_ref)

    box_acc = None
    for s in range(_NSLAB):
        aw = anch_ref[2 * (_NSLAB * i + s)]
        ah = anch_ref[2 * (_NSLAB * i + s) + 1]
        bx, ob, cl = _slab_sums(pred_ref[s], tgt_ref[s], mask_ref[s], aw, ah)
        mk = mask_ref[s]
        if box_acc is None:
            box_acc, obj_acc, cls_acc, msk_acc = bx, ob, cl, mk
        else:
            box_acc += bx
            obj_acc += ob
            cls_acc += cl
            msk_acc += mk

    acc_ref[0:1, :] += box_acc
    acc_ref[1:2, :] += obj_acc
    acc_ref[2:3, :] += cls_acc
    acc_ref[3:4, :] += msk_acc

    @pl.when(i == n - 1)
    def _():
        out_ref[0] = jnp.sum(acc_ref[0:1, :])
        out_ref[1] = jnp.sum(acc_ref[1:2, :])
        out_ref[2] = jnp.sum(acc_ref[2:3, :])
        out_ref[3] = jnp.sum(acc_ref[3:4, :])


def kernel(predictions, targets_in_grid, targets_masks, anchors):
    B, A, F, H, W = predictions.shape
    HW = H * W
    pred = predictions.reshape(B * A, F, HW)
    tgt = targets_in_grid.reshape(B * A, F, HW)
    mask = targets_masks.reshape(B * A, 1, HW).astype(jnp.float32)
    anch = jnp.broadcast_to(anchors[None, :, 2:4], (B, A, 2)).reshape(-1)

    grid_spec = pltpu.PrefetchScalarGridSpec(
        num_scalar_prefetch=1,
        grid=(B * A // _NSLAB,),
        in_specs=[
            pl.BlockSpec((_NSLAB, F, HW), lambda i, anch_ref: (i, 0, 0)),
            pl.BlockSpec((_NSLAB, F, HW), lambda i, anch_ref: (i, 0, 0)),
            pl.BlockSpec((_NSLAB, 1, HW), lambda i, anch_ref: (i, 0, 0)),
        ],
        out_specs=pl.BlockSpec(memory_space=pltpu.SMEM),
        scratch_shapes=[pltpu.VMEM((8, HW), jnp.float32)],
    )
    sums = pl.pallas_call(
        _loss_kernel,
        grid_spec=grid_spec,
        out_shape=jax.ShapeDtypeStruct((4,), jnp.float32),
    )(anch, pred, tgt, mask)

    n_pos = jnp.maximum(sums[3], 1.0)
    n_obj = jnp.float32(B * A * H * W)
    return sums[0] / n_pos + sums[1] / n_obj + sums[2] / (n_pos * (F - 5))


# speedup vs baseline: 1.0096x; 1.0096x over previous
import jax
import jax.numpy as jnp
import numpy as np
from jax.experimental import pallas as pl
from jax.experimental.pallas import tpu as pltpu

_G = 2.0
_EPS = 1e-7
_NSLAB = 4


def _atan(x):
    s = jnp.sign(x)
    ax = jnp.abs(x)
    big = ax > 2.414213562373095
    mid = ax > 0.4142135623730951
    xr = jnp.where(big, -1.0 / jnp.maximum(ax, 1e-30),
                   jnp.where(mid, (ax - 1.0) / (ax + 1.0), ax))
    z = xr * xr
    p = ((8.05374449538e-2 * z - 1.38776856032e-1) * z
         + 1.99777106478e-1) * z - 3.33329491539e-1
    y = p * z * xr + xr
    y = y + jnp.where(big, np.float32(np.pi / 2),
                      jnp.where(mid, np.float32(np.pi / 4), 0.0))
    return s * y


def _slab_sums(pred, tgt, mask, aw, ah):
    log2e = np.float32(1.4426950408889634)
    ln2 = np.float32(0.6931471805599453)
    sp = jnp.log2(1.0 + jnp.exp2(jnp.abs(pred) * (-log2e))) * ln2
    bce = jnp.maximum(pred, 0.0) - pred * tgt + sp
    colsum = jnp.sum(bce, axis=0, keepdims=True)
    head = bce[0:8, :]
    obj_vec = head[4:5, :]
    h04 = jnp.sum(head[0:5, :], axis=0, keepdims=True)
    cls_vec = (colsum - h04) * mask

    sig = jax.nn.sigmoid(pred[0:4, :])
    px = sig[0:1, :] * _G - (_G - 1.0) / 2.0
    py = sig[1:2, :] * _G - (_G - 1.0) / 2.0
    pw = (sig[2:3, :] * _G) ** 2 * aw
    ph = (sig[3:4, :] * _G) ** 2 * ah
    tx = tgt[0:1, :]
    ty = tgt[1:2, :]
    tw = tgt[2:3, :]
    th = tgt[3:4, :]

    b1x1 = px - pw * 0.5
    b1x2 = px + pw * 0.5
    b1y1 = py - ph * 0.5
    b1y2 = py + ph * 0.5
    b2x1 = tx - tw * 0.5
    b2x2 = tx + tw * 0.5
    b2y1 = ty - th * 0.5
    b2y2 = ty + th * 0.5
    inter = (jnp.maximum(jnp.minimum(b1x2, b2x2) - jnp.maximum(b1x1, b2x1), 0.0)
             * jnp.maximum(jnp.minimum(b1y2, b2y2) - jnp.maximum(b1y1, b2y1), 0.0))
    union = pw * ph + tw * th - inter + _EPS
    iou = inter / union
    cw = jnp.maximum(b1x2, b2x2) - jnp.minimum(b1x1, b2x1)
    ch = jnp.maximum(b1y2, b2y2) - jnp.minimum(b1y1, b2y1)
    c2 = cw * cw + ch * ch + _EPS
    rho2 = (tx - px) ** 2 + (ty - py) ** 2
    ra = tw / (th + _EPS)
    rb = pw / (ph + _EPS)
    v = (4.0 / np.pi ** 2) * _atan((ra - rb) / (1.0 + ra * rb)) ** 2
    alpha = v / (v - iou + 1.0 + _EPS)
    ciou = iou - (rho2 / c2 + v * alpha)
    box_vec = (1.0 - ciou) * mask
    return box_vec, obj_vec, cls_vec


def _loss_kernel(anch_ref, pred_hbm, tgt_hbm, mask_hbm, out_ref,
                 pred_buf, tgt_buf, mask_buf, acc_ref, sems):
    i = pl.program_id(0)
    n = pl.num_programs(0)

    def _copies(blk, slot):
        base = blk * _NSLAB
        return (
            pltpu.make_async_copy(pred_hbm.at[pl.ds(base, _NSLAB)],
                                  pred_buf.at[slot], sems.at[slot, 0]),
            pltpu.make_async_copy(tgt_hbm.at[pl.ds(base, _NSLAB)],
                                  tgt_buf.at[slot], sems.at[slot, 1]),
            pltpu.make_async_copy(mask_hbm.at[pl.ds(base, _NSLAB)],
                                  mask_buf.at[slot], sems.at[slot, 2]),
        )

    @pl.when(i == 0)
    def _():
        acc_ref[...] = jnp.zeros_like(acc_ref)
        for c in _copies(0, 0):
            c.start()

    @pl.when(i + 1 < n)
    def _():
        for c in _copies(i + 1, (i + 1) % 2):
            c.start()

    slot = jax.lax.rem(i, 2)
    for c in _copies(i, slot):
        c.wait()

    box_acc = None
    for s in range(_NSLAB):
        aw = anch_ref[2 * (_NSLAB * i + s)]
        ah = anch_ref[2 * (_NSLAB * i + s) + 1]
        bx, ob, cl = _slab_sums(pred_buf[slot, s], tgt_buf[slot, s],
                                mask_buf[slot, s], aw, ah)
        mk = mask_buf[slot, s]
        if box_acc is None:
            box_acc, obj_acc, cls_acc, msk_acc = bx, ob, cl, mk
        else:
            box_acc += bx
            obj_acc += ob
            cls_acc += cl
            msk_acc += mk

    acc_ref[0:1, :] += box_acc
    acc_ref[1:2, :] += obj_acc
    acc_ref[2:3, :] += cls_acc
    acc_ref[3:4, :] += msk_acc

    @pl.when(i == n - 1)
    def _():
        out_ref[0] = jnp.sum(acc_ref[0:1, :])
        out_ref[1] = jnp.sum(acc_ref[1:2, :])
        out_ref[2] = jnp.sum(acc_ref[2:3, :])
        out_ref[3] = jnp.sum(acc_ref[3:4, :])


def kernel(predictions, targets_in_grid, targets_masks, anchors):
    B, A, F, H, W = predictions.shape
    HW = H * W
    pred = predictions.reshape(B * A, F, HW)
    tgt = targets_in_grid.reshape(B * A, F, HW)
    mask = targets_masks.reshape(B * A, 1, HW).astype(jnp.float32)
    anch = jnp.broadcast_to(anchors[None, :, 2:4], (B, A, 2)).reshape(-1)

    grid_spec = pltpu.PrefetchScalarGridSpec(
        num_scalar_prefetch=1,
        grid=(B * A // _NSLAB,),
        in_specs=[
            pl.BlockSpec(memory_space=pltpu.MemorySpace.HBM),
            pl.BlockSpec(memory_space=pltpu.MemorySpace.HBM),
            pl.BlockSpec(memory_space=pltpu.MemorySpace.HBM),
        ],
        out_specs=pl.BlockSpec(memory_space=pltpu.SMEM),
        scratch_shapes=[
            pltpu.VMEM((2, _NSLAB, F, HW), jnp.float32),
            pltpu.VMEM((2, _NSLAB, F, HW), jnp.float32),
            pltpu.VMEM((2, _NSLAB, 1, HW), jnp.float32),
            pltpu.VMEM((8, HW), jnp.float32),
            pltpu.SemaphoreType.DMA((2, 3)),
        ],
    )
    sums = pl.pallas_call(
        _loss_kernel,
        grid_spec=grid_spec,
        out_shape=jax.ShapeDtypeStruct((4,), jnp.float32),
    )(anch, pred, tgt, mask)

    n_pos = jnp.maximum(sums[3], 1.0)
    n_obj = jnp.float32(B * A * H * W)
    return sums[0] / n_pos + sums[1] / n_obj + sums[2] / (n_pos * (F - 5))


# speedup vs baseline: 1.0250x; 1.0153x over previous
import jax
import jax.numpy as jnp
import numpy as np
from jax.experimental import pallas as pl
from jax.experimental.pallas import tpu as pltpu

_G = 2.0
_EPS = 1e-7
_NSLAB = 4


def _atan(x):
    s = jnp.sign(x)
    ax = jnp.abs(x)
    big = ax > 2.414213562373095
    mid = ax > 0.4142135623730951
    xr = jnp.where(big, -1.0 / jnp.maximum(ax, 1e-30),
                   jnp.where(mid, (ax - 1.0) / (ax + 1.0), ax))
    z = xr * xr
    p = ((8.05374449538e-2 * z - 1.38776856032e-1) * z
         + 1.99777106478e-1) * z - 3.33329491539e-1
    y = p * z * xr + xr
    y = y + jnp.where(big, np.float32(np.pi / 2),
                      jnp.where(mid, np.float32(np.pi / 4), 0.0))
    return s * y


def _slab_sums(pred, tgt, mask, wmat, aw, ah):
    log2e = np.float32(1.4426950408889634)
    ln2 = np.float32(0.6931471805599453)
    sp = jnp.log2(1.0 + jnp.exp2(jnp.abs(pred) * (-log2e))) * ln2
    bce = jnp.maximum(pred, 0.0) - pred * tgt + sp
    red = jax.lax.dot_general(wmat, bce, (((1,), (0,)), ((), ())),
                              preferred_element_type=jnp.float32)
    colsum = red[0:1, :]
    h04 = red[1:2, :]
    obj_vec = red[2:3, :]
    cls_vec = (colsum - h04) * mask

    sig4 = jax.nn.sigmoid(pred[0:4, :])
    g2 = sig4 * _G
    rowi = jax.lax.broadcasted_iota(jnp.int32, g2.shape, 0)
    awh = jnp.where(rowi == 2, aw, ah)
    dec = jnp.where(rowi < 2, g2 - (_G - 1.0) / 2.0, g2 * g2 * awh)
    xy = dec[0:2, :]
    wh = dec[2:4, :]
    txy = tgt[0:2, :]
    twh = tgt[2:4, :]

    b1lo = xy - wh * 0.5
    b1hi = xy + wh * 0.5
    b2lo = txy - twh * 0.5
    b2hi = txy + twh * 0.5
    iwh = jnp.maximum(jnp.minimum(b1hi, b2hi) - jnp.maximum(b1lo, b2lo), 0.0)
    inter = iwh[0:1, :] * iwh[1:2, :]
    cwh = jnp.maximum(b1hi, b2hi) - jnp.minimum(b1lo, b2lo)
    c2v = cwh * cwh
    c2 = c2v[0:1, :] + c2v[1:2, :] + _EPS
    dxy = txy - xy
    d2 = dxy * dxy
    rho2 = d2[0:1, :] + d2[1:2, :]
    pa = wh[0:1, :] * wh[1:2, :]
    ta = twh[0:1, :] * twh[1:2, :]
    union = pa + ta - inter + _EPS
    iou = inter / union
    ra = twh[0:1, :] / (twh[1:2, :] + _EPS)
    rb = wh[0:1, :] / (wh[1:2, :] + _EPS)
    v = (4.0 / np.pi ** 2) * _atan((ra - rb) / (1.0 + ra * rb)) ** 2
    alpha = v / (v - iou + 1.0 + _EPS)
    ciou = iou - (rho2 / c2 + v * alpha)
    box_vec = (1.0 - ciou) * mask
    return box_vec, obj_vec, cls_vec


def _loss_kernel(anch_ref, pred_hbm, tgt_hbm, mask_hbm, wmat_ref, out_ref,
                 pred_buf, tgt_buf, mask_buf, acc_ref, sems):
    i = pl.program_id(0)
    n = pl.num_programs(0)

    def _copies(blk, slot):
        base = blk * _NSLAB
        return (
            pltpu.make_async_copy(pred_hbm.at[pl.ds(base, _NSLAB)],
                                  pred_buf.at[slot], sems.at[slot, 0]),
            pltpu.make_async_copy(tgt_hbm.at[pl.ds(base, _NSLAB)],
                                  tgt_buf.at[slot], sems.at[slot, 1]),
            pltpu.make_async_copy(mask_hbm.at[pl.ds(base, _NSLAB)],
                                  mask_buf.at[slot], sems.at[slot, 2]),
        )

    @pl.when(i == 0)
    def _():
        acc_ref[...] = jnp.zeros_like(acc_ref)
        for c in _copies(0, 0):
            c.start()

    @pl.when(i + 1 < n)
    def _():
        for c in _copies(i + 1, (i + 1) % 2):
            c.start()

    slot = jax.lax.rem(i, 2)
    for c in _copies(i, slot):
        c.wait()

    wmat = wmat_ref[...]
    box_acc = None
    for s in range(_NSLAB):
        aw = anch_ref[2 * (_NSLAB * i + s)]
        ah = anch_ref[2 * (_NSLAB * i + s) + 1]
        bx, ob, cl = _slab_sums(pred_buf[slot, s], tgt_buf[slot, s],
                                mask_buf[slot, s], wmat, aw, ah)
        mk = mask_buf[slot, s]
        if box_acc is None:
            box_acc, obj_acc, cls_acc, msk_acc = bx, ob, cl, mk
        else:
            box_acc += bx
            obj_acc += ob
            cls_acc += cl
            msk_acc += mk

    acc_ref[0:1, :] += box_acc
    acc_ref[1:2, :] += obj_acc
    acc_ref[2:3, :] += cls_acc
    acc_ref[3:4, :] += msk_acc

    @pl.when(i == n - 1)
    def _():
        out_ref[0] = jnp.sum(acc_ref[0:1, :])
        out_ref[1] = jnp.sum(acc_ref[1:2, :])
        out_ref[2] = jnp.sum(acc_ref[2:3, :])
        out_ref[3] = jnp.sum(acc_ref[3:4, :])


def kernel(predictions, targets_in_grid, targets_masks, anchors):
    B, A, F, H, W = predictions.shape
    HW = H * W
    pred = predictions.reshape(B * A, F, HW)
    tgt = targets_in_grid.reshape(B * A, F, HW)
    mask = targets_masks.reshape(B * A, 1, HW).astype(jnp.float32)
    anch = jnp.broadcast_to(anchors[None, :, 2:4], (B, A, 2)).reshape(-1)
    wnp = np.zeros((8, F), dtype=np.float32)
    wnp[0, :] = 1.0
    wnp[1, 0:5] = 1.0
    wnp[2, 4] = 1.0
    wmat = jnp.asarray(wnp)

    grid_spec = pltpu.PrefetchScalarGridSpec(
        num_scalar_prefetch=1,
        grid=(B * A // _NSLAB,),
        in_specs=[
            pl.BlockSpec(memory_space=pltpu.MemorySpace.HBM),
            pl.BlockSpec(memory_space=pltpu.MemorySpace.HBM),
            pl.BlockSpec(memory_space=pltpu.MemorySpace.HBM),
            pl.BlockSpec((8, F), lambda i, anch_ref: (0, 0)),
        ],
        out_specs=pl.BlockSpec(memory_space=pltpu.SMEM),
        scratch_shapes=[
            pltpu.VMEM((2, _NSLAB, F, HW), jnp.float32),
            pltpu.VMEM((2, _NSLAB, F, HW), jnp.float32),
            pltpu.VMEM((2, _NSLAB, 1, HW), jnp.float32),
            pltpu.VMEM((8, HW), jnp.float32),
            pltpu.SemaphoreType.DMA((2, 3)),
        ],
    )
    sums = pl.pallas_call(
        _loss_kernel,
        grid_spec=grid_spec,
        out_shape=jax.ShapeDtypeStruct((4,), jnp.float32),
    )(anch, pred, tgt, mask, wmat)

    n_pos = jnp.maximum(sums[3], 1.0)
    n_obj = jnp.float32(B * A * H * W)
    return sums[0] / n_pos + sums[1] / n_obj + sums[2] / (n_pos * (F - 5))


# speedup vs baseline: 1.0274x; 1.0023x over previous
import jax
import jax.numpy as jnp
import numpy as np
from jax.experimental import pallas as pl
from jax.experimental.pallas import tpu as pltpu

_G = 2.0
_EPS = 1e-7
_NSLAB = 8


def _atan(x):
    s = jnp.sign(x)
    ax = jnp.abs(x)
    big = ax > 2.414213562373095
    mid = ax > 0.4142135623730951
    xr = jnp.where(big, -1.0 / jnp.maximum(ax, 1e-30),
                   jnp.where(mid, (ax - 1.0) / (ax + 1.0), ax))
    z = xr * xr
    p = ((8.05374449538e-2 * z - 1.38776856032e-1) * z
         + 1.99777106478e-1) * z - 3.33329491539e-1
    y = p * z * xr + xr
    y = y + jnp.where(big, np.float32(np.pi / 2),
                      jnp.where(mid, np.float32(np.pi / 4), 0.0))
    return s * y


def _slab_sums(pred, tgt, mask, wmat, aw, ah):
    log2e = np.float32(1.4426950408889634)
    ln2 = np.float32(0.6931471805599453)
    sp = jnp.log2(1.0 + jnp.exp2(jnp.abs(pred) * (-log2e))) * ln2
    bce = jnp.maximum(pred, 0.0) - pred * tgt + sp
    red = jax.lax.dot_general(wmat, bce, (((1,), (0,)), ((), ())),
                              preferred_element_type=jnp.float32)
    colsum = red[0:1, :]
    h04 = red[1:2, :]
    obj_vec = red[2:3, :]
    cls_vec = (colsum - h04) * mask

    sig4 = jax.nn.sigmoid(pred[0:4, :])
    g2 = sig4 * _G
    rowi = jax.lax.broadcasted_iota(jnp.int32, g2.shape, 0)
    awh = jnp.where(rowi == 2, aw, ah)
    dec = jnp.where(rowi < 2, g2 - (_G - 1.0) / 2.0, g2 * g2 * awh)
    xy = dec[0:2, :]
    wh = dec[2:4, :]
    txy = tgt[0:2, :]
    twh = tgt[2:4, :]

    b1lo = xy - wh * 0.5
    b1hi = xy + wh * 0.5
    b2lo = txy - twh * 0.5
    b2hi = txy + twh * 0.5
    iwh = jnp.maximum(jnp.minimum(b1hi, b2hi) - jnp.maximum(b1lo, b2lo), 0.0)
    inter = iwh[0:1, :] * iwh[1:2, :]
    cwh = jnp.maximum(b1hi, b2hi) - jnp.minimum(b1lo, b2lo)
    c2v = cwh * cwh
    c2 = c2v[0:1, :] + c2v[1:2, :] + _EPS
    dxy = txy - xy
    d2 = dxy * dxy
    rho2 = d2[0:1, :] + d2[1:2, :]
    pa = wh[0:1, :] * wh[1:2, :]
    ta = twh[0:1, :] * twh[1:2, :]
    union = pa + ta - inter + _EPS
    iou = inter / union
    ra = twh[0:1, :] / (twh[1:2, :] + _EPS)
    rb = wh[0:1, :] / (wh[1:2, :] + _EPS)
    v = (4.0 / np.pi ** 2) * _atan((ra - rb) / (1.0 + ra * rb)) ** 2
    alpha = v / (v - iou + 1.0 + _EPS)
    ciou = iou - (rho2 / c2 + v * alpha)
    box_vec = (1.0 - ciou) * mask
    return box_vec, obj_vec, cls_vec


def _loss_kernel(anch_ref, pred_hbm, tgt_hbm, mask_hbm, wmat_ref, out_ref,
                 pred_buf, tgt_buf, mask_buf, acc_ref, sems):
    i = pl.program_id(0)
    n = pl.num_programs(0)

    def _copies(blk, slot):
        base = blk * _NSLAB
        return (
            pltpu.make_async_copy(pred_hbm.at[pl.ds(base, _NSLAB)],
                                  pred_buf.at[slot], sems.at[slot, 0]),
            pltpu.make_async_copy(tgt_hbm.at[pl.ds(base, _NSLAB)],
                                  tgt_buf.at[slot], sems.at[slot, 1]),
            pltpu.make_async_copy(mask_hbm.at[pl.ds(base, _NSLAB)],
                                  mask_buf.at[slot], sems.at[slot, 2]),
        )

    @pl.when(i == 0)
    def _():
        acc_ref[...] = jnp.zeros_like(acc_ref)
        for c in _copies(0, 0):
            c.start()

    @pl.when(i + 1 < n)
    def _():
        for c in _copies(i + 1, (i + 1) % 2):
            c.start()

    slot = jax.lax.rem(i, 2)
    for c in _copies(i, slot):
        c.wait()

    wmat = wmat_ref[...]
    box_acc = None
    for s in range(_NSLAB):
        aw = anch_ref[2 * (_NSLAB * i + s)]
        ah = anch_ref[2 * (_NSLAB * i + s) + 1]
        bx, ob, cl = _slab_sums(pred_buf[slot, s], tgt_buf[slot, s],
                                mask_buf[slot, s], wmat, aw, ah)
        mk = mask_buf[slot, s]
        if box_acc is None:
            box_acc, obj_acc, cls_acc, msk_acc = bx, ob, cl, mk
        else:
            box_acc += bx
            obj_acc += ob
            cls_acc += cl
            msk_acc += mk

    acc_ref[0:1, :] += box_acc
    acc_ref[1:2, :] += obj_acc
    acc_ref[2:3, :] += cls_acc
    acc_ref[3:4, :] += msk_acc

    @pl.when(i == n - 1)
    def _():
        out_ref[0] = jnp.sum(acc_ref[0:1, :])
        out_ref[1] = jnp.sum(acc_ref[1:2, :])
        out_ref[2] = jnp.sum(acc_ref[2:3, :])
        out_ref[3] = jnp.sum(acc_ref[3:4, :])


def kernel(predictions, targets_in_grid, targets_masks, anchors):
    B, A, F, H, W = predictions.shape
    HW = H * W
    pred = predictions.reshape(B * A, F, HW)
    tgt = targets_in_grid.reshape(B * A, F, HW)
    mask = targets_masks.reshape(B * A, 1, HW).astype(jnp.float32)
    anch = jnp.broadcast_to(anchors[None, :, 2:4], (B, A, 2)).reshape(-1)
    wnp = np.zeros((8, F), dtype=np.float32)
    wnp[0, :] = 1.0
    wnp[1, 0:5] = 1.0
    wnp[2, 4] = 1.0
    wmat = jnp.asarray(wnp)

    grid_spec = pltpu.PrefetchScalarGridSpec(
        num_scalar_prefetch=1,
        grid=(B * A // _NSLAB,),
        in_specs=[
            pl.BlockSpec(memory_space=pltpu.MemorySpace.HBM),
            pl.BlockSpec(memory_space=pltpu.MemorySpace.HBM),
            pl.BlockSpec(memory_space=pltpu.MemorySpace.HBM),
            pl.BlockSpec((8, F), lambda i, anch_ref: (0, 0)),
        ],
        out_specs=pl.BlockSpec(memory_space=pltpu.SMEM),
        scratch_shapes=[
            pltpu.VMEM((2, _NSLAB, F, HW), jnp.float32),
            pltpu.VMEM((2, _NSLAB, F, HW), jnp.float32),
            pltpu.VMEM((2, _NSLAB, 1, HW), jnp.float32),
            pltpu.VMEM((8, HW), jnp.float32),
            pltpu.SemaphoreType.DMA((2, 3)),
        ],
    )
    sums = pl.pallas_call(
        _loss_kernel,
        grid_spec=grid_spec,
        out_shape=jax.ShapeDtypeStruct((4,), jnp.float32),
    )(anch, pred, tgt, mask, wmat)

    n_pos = jnp.maximum(sums[3], 1.0)
    n_obj = jnp.float32(B * A * H * W)
    return sums[0] / n_pos + sums[1] / n_obj + sums[2] / (n_pos * (F - 5))
